# Initial kernel scaffold; baseline (speedup 1.0000x reference)
import jax
import jax.numpy as jnp
from jax.experimental import pallas as pl


def kernel(x, token_table, pos_table):
    raise NotImplementedError("write your pallas kernel here")


# speedup vs baseline: 1.4251x; 1.4251x over previous
import functools

import jax
import jax.numpy as jnp
from jax import lax
from jax.experimental import pallas as pl
from jax.experimental.pallas import tpu as pltpu
from jax.experimental.pallas import tpu_sc as plsc

VOCAB = 1000000
CTX = 200
DIM = 32
BATCH = 4096
BL = BATCH * CTX

NC = 2
NS = 16
NW = NC * NS
PER_W = BL // NW
NB = 8
CH = NB * CTX
G = PER_W // CH


def _body(x_hbm, tok_hbm, pos_hbm, out_hbm, idx_v, rows_v, pos_v, sem):
    wid = lax.axis_index("s") * NC + lax.axis_index("c")
    base = wid * PER_W

    pltpu.sync_copy(pos_hbm, pos_v)

    for g in range(G):
        off = base + g * CH
        pltpu.sync_copy(x_hbm.at[pl.ds(off, CH)], idx_v)
        pltpu.async_copy(tok_hbm.at[idx_v], rows_v, sem).wait()

        def add_l(l, _):
            p0 = pos_v[l, pl.ds(0, 16)]
            p1 = pos_v[l, pl.ds(16, 16)]
            for sb in range(NB):
                r = sb * CTX + l
                rows_v[r, pl.ds(0, 16)] = rows_v[r, pl.ds(0, 16)] + p0
                rows_v[r, pl.ds(16, 16)] = rows_v[r, pl.ds(16, 16)] + p1
            return 0

        lax.fori_loop(0, CTX, add_l, 0, unroll=2)

        pltpu.sync_copy(rows_v, out_hbm.at[pl.ds(off, CH)])


@jax.jit
def _embed(x_flat, token_table, pos_table):
    mesh = plsc.VectorSubcoreMesh(core_axis_name="c", subcore_axis_name="s")
    return pl.kernel(
        _body,
        out_type=jax.ShapeDtypeStruct((BL, DIM), jnp.float32),
        mesh=mesh,
        scratch_types=[
            pltpu.VMEM((CH,), jnp.int32),
            pltpu.VMEM((CH, DIM), jnp.float32),
            pltpu.VMEM((CTX, DIM), jnp.float32),
            pltpu.SemaphoreType.DMA,
        ],
        compiler_params=pltpu.CompilerParams(use_tc_tiling_on_sc=False),
    )(x_flat, token_table, pos_table)


def kernel(x, token_table, pos_table):
    x_flat = jnp.reshape(x, (-1,)).astype(jnp.int32)
    out = _embed(x_flat, token_table, pos_table)
    return jnp.reshape(out, (BATCH, CTX, DIM))
